# baseline (speedup 1.0000x reference)
import functools

import jax
import jax.numpy as jnp
from jax.experimental import pallas as pl


def _ranks_kernel(qb_ref, kb_ref, qf_ref, kf_ref, rq_ref, rk_ref, *, rb, ch):
    s = qf_ref.shape[1]
    i = pl.program_id(1)
    qb = qb_ref[0][:, None, :]
    kb = kb_ref[0][:, None, :]
    rows = i * rb + jax.lax.broadcasted_iota(jnp.int32, (rb, 1), 0).reshape(
        rb, 1, 1
    )

    def body(ci, accs):
        rq, rk = accs
        qc = qf_ref[0, pl.ds(ci * ch, ch), :][None, :, :]
        kc = kf_ref[0, pl.ds(ci * ch, ch), :][None, :, :]
        cids = ci * ch + jax.lax.broadcasted_iota(jnp.int32, (ch, 1), 0).reshape(
            1, ch, 1
        )
        idx_lt = cids < rows
        one = jnp.int32(1)
        zero = jnp.int32(0)
        cnt_q = jnp.where((qc < qb) | ((qc == qb) & idx_lt), one, zero)
        cnt_k = jnp.where((kc < kb) | ((kc == kb) & idx_lt), one, zero)
        rq = rq + jnp.sum(cnt_q, axis=1)
        rk = rk + jnp.sum(cnt_k, axis=1)
        return rq, rk

    d = qb_ref.shape[-1]
    z = jnp.zeros((rb, d), jnp.int32)
    rq, rk = jax.lax.fori_loop(0, s // ch, body, (z, z))
    rq_ref[0] = rq
    rk_ref[0] = rk


def _pair_assemble_kernel(
    qb_ref, rqb_ref, kf_ref, rkf_ref, mask_ref, out_ref, *, rb, ch
):
    s = kf_ref.shape[1]
    d = kf_ref.shape[2]
    rq = rqb_ref[0][:, None, :]

    def body(ci, accs):
        kv, cc = accs
        kc = kf_ref[0, pl.ds(ci * ch, ch), :][None, :, :]
        rkc = rkf_ref[0, pl.ds(ci * ch, ch), :][None, :, :]
        cids = ci * ch + jax.lax.broadcasted_iota(jnp.int32, (ch, 1), 0).reshape(
            1, ch, 1
        )
        eq = rkc == rq
        kv = kv + jnp.sum(jnp.where(eq, kc, 0.0), axis=1)
        cc = cc + jnp.sum(jnp.where(eq, cids, 0), axis=1)
        return kv, cc

    kv = jnp.zeros((rb, d), jnp.float32)
    cc = jnp.zeros((rb, d), jnp.int32)
    kv, cc = jax.lax.fori_loop(0, s // ch, body, (kv, cc))
    v = jnp.exp(-jnp.square(qb_ref[0] - kv)) * (1.0 / d)
    cols = jax.lax.broadcasted_iota(jnp.int32, (rb, s), 1)
    acc = jnp.zeros((rb, s), jnp.float32)
    for f in range(d):
        acc = acc + jnp.where(cc[:, f][:, None] == cols, v[:, f][:, None], 0.0)
    out_ref[0] = jnp.where(mask_ref[0], 0.0, acc)


def kernel(q, k, attn_mask):
    mask_shape = attn_mask.shape
    s = q.shape[-2]
    d = q.shape[-1]
    b = 1
    for dim in q.shape[:-2]:
        b *= dim
    qf = q.reshape(b, s, d)
    kf = k.reshape(b, s, d)
    mf = attn_mask.reshape(b, s, s)

    rb = min(128, s)
    ch = min(128, s)
    nb = s // rb

    full_spec = pl.BlockSpec((1, s, d), lambda bi, i: (bi, 0, 0))
    blk_spec = pl.BlockSpec((1, rb, d), lambda bi, i: (bi, i, 0))

    rq, rk = pl.pallas_call(
        functools.partial(_ranks_kernel, rb=rb, ch=ch),
        grid=(b, nb),
        in_specs=[blk_spec, blk_spec, full_spec, full_spec],
        out_specs=[blk_spec, blk_spec],
        out_shape=[
            jax.ShapeDtypeStruct((b, s, d), jnp.int32),
            jax.ShapeDtypeStruct((b, s, d), jnp.int32),
        ],
    )(qf, kf, qf, kf)

    row_spec = pl.BlockSpec((1, rb, s), lambda bi, i: (bi, i, 0))
    p = pl.pallas_call(
        functools.partial(_pair_assemble_kernel, rb=rb, ch=ch),
        grid=(b, nb),
        in_specs=[blk_spec, blk_spec, full_spec, full_spec, row_spec],
        out_specs=row_spec,
        out_shape=jax.ShapeDtypeStruct((b, s, s), jnp.float32),
    )(qf, rq, kf, rk, mf)

    return p.reshape(mask_shape)
